# baseline (speedup 1.0000x reference)
import functools

import jax
import jax.numpy as jnp
import numpy as np
from jax import lax
from jax.experimental import pallas as pl
from jax.experimental.pallas import tpu as pltpu
from jax.experimental.pallas import tpu_sc as plsc

THR = 0.2
BORDER_RM = 2
NUM_MATCHES = 5000
B, H0C, W0C, H1C, W1C = 2, 60, 80, 60, 80
L = H0C * W0C
S = H1C * W1C
RB = 800
NR = L // RB
NROWS = B * L

NT = 16
NTA = 6
CHUNK = 800
NV = CHUNK // 16
OUT_PAD = 5000
OUT_PT = 320
CSIZE = 5248
TRASH = OUT_PAD + 16

_THR_GE = float(np.nextafter(np.float32(THR), np.float32(1.0)))


def _merged_body(conf_ref, jio_ref, cmax_ref, rm_ref, j1_ref, jl_ref):
    r = pl.program_id(0)
    x = conf_ref[0]
    rm = jnp.max(x, axis=1, keepdims=True)
    pmax = jnp.max(conf_ref[...], axis=1, keepdims=True)

    @pl.when(r == 0)
    def _():
        cmax_ref[...] = pmax

    @pl.when(r != 0)
    def _():
        cmax_ref[...] = jnp.maximum(cmax_ref[...], pmax)

    ge = x >= rm
    jio = jio_ref[0]
    j1 = jnp.min(jnp.where(ge, jio, jnp.float32(S)), axis=1)
    jl = jnp.max(jnp.where(ge, jio, jnp.float32(-1)), axis=1)
    rm_ref[...] = rm.reshape(1, 1, RB)
    j1_ref[...] = j1.astype(jnp.int32).reshape(1, 1, RB)
    jl_ref[...] = jl.astype(jnp.int32).reshape(1, 1, RB)


def _sc_rows(bat, wid, rm_hbm, j1_hbm, jl_hbm, bori_hbm, cm_hbm, borj_hbm,
             conf_hbm, rm_c, j1_c, jl_c, bori_c, cm_v, borj_v,
             flags_v, ranks_v, fj_v, mrow_v, tie_v, row_v, dsem):
    src = pl.ds(wid * CHUNK, CHUNK)
    cps = [pltpu.async_copy(rm_hbm.at[wid, 0], rm_c, dsem),
           pltpu.async_copy(j1_hbm.at[wid, 0], j1_c, dsem),
           pltpu.async_copy(jl_hbm.at[wid, 0], jl_c, dsem),
           pltpu.async_copy(bori_hbm.at[src], bori_c, dsem),
           pltpu.async_copy(cm_hbm.at[0, 0], cm_v, dsem),
           pltpu.async_copy(borj_hbm, borj_v, dsem)]
    for cp in cps:
        cp.wait()

    ntie = jnp.int32(0)
    for k in range(NV):
        sl = pl.ds(k * 16, 16)
        rmv = rm_c[sl]
        j1v = j1_c[sl]
        jlv = jl_c[sl]
        c1 = plsc.load_gather(cm_v, [j1v])
        bj1 = plsc.load_gather(borj_v, [j1v])
        rowok = (bori_c[sl] > 0) & (rmv >= _THR_GE)
        tie = rowok & (jlv > j1v)
        okrow = rowok & (jlv == j1v) & (c1 == rmv) & (bj1 > 0)
        flags_v[sl] = okrow.astype(jnp.int32)
        fj_v[sl] = jnp.where(okrow, j1v, 0)
        mrow_v[sl] = jnp.where(okrow, rmv, 0.0)
        ti = tie.astype(jnp.int32)
        plsc.store_scatter(tie_v, [ntie + plsc.cumsum(ti) - ti],
                           k * 16 + lax.iota(jnp.int32, 16), mask=tie)
        ntie = ntie + jnp.sum(ti)

    def _refine(t, carry):
        r = plsc.load_gather(tie_v, [jnp.full((16,), t, jnp.int32)])[0]
        grow = bat * L + wid * CHUNK + r
        pltpu.sync_copy(conf_hbm.at[grow], row_v)
        rms = plsc.load_gather(rm_c, [jnp.full((16,), r, jnp.int32)])[0]
        rmf = jnp.full((16,), rms, jnp.float32)

        def _scan(k, vmin):
            cv = row_v[pl.ds(k * 16, 16)]
            cmv = cm_v[pl.ds(k * 16, 16)]
            bjv = borj_v[pl.ds(k * 16, 16)]
            jv = k * 16 + lax.iota(jnp.int32, 16)
            hit = (cv == rmf) & (cmv == rmf) & (bjv > 0)
            return jnp.minimum(vmin, jnp.where(hit, jv, S))

        vmin = lax.fori_loop(0, S // 16, _scan,
                             jnp.full((16,), S, jnp.int32))
        fjs = jnp.min(vmin)
        found = fjs < S
        base = (r // 16) * 16
        eq = lax.iota(jnp.int32, 16) == (r - base)
        bsl = pl.ds(base, 16)
        flags_v[bsl] = jnp.where(eq, found.astype(jnp.int32), flags_v[bsl])
        fnd = eq & found
        fj_v[bsl] = jnp.where(fnd, fjs, fj_v[bsl])
        mrow_v[bsl] = jnp.where(fnd, rms, mrow_v[bsl])
        return carry

    lax.fori_loop(0, ntie, _refine, jnp.int32(0))

    cnt = jnp.int32(0)
    for k in range(NV):
        sl = pl.ds(k * 16, 16)
        f = flags_v[sl]
        ranks_v[sl] = cnt + (plsc.cumsum(f) - f)
        cnt = cnt + jnp.sum(f)
    return cnt


def _sc_scatter(wid, base, flags_v, ranks_v, slots_v, vals_v, compact_sp,
                dsem):
    trash = TRASH + wid
    for k in range(56):
        row, col = k // 8, (k % 8) * 16
        if k < NV:
            f = flags_v[pl.ds(k * 16, 16)]
            slot = base + ranks_v[pl.ds(k * 16, 16)]
            ok = (f > 0) & (slot < OUT_PAD)
            slots_v[row, pl.ds(col, 16)] = jnp.where(ok, slot, trash)
            vals_v[row, pl.ds(col, 16)] = (
                wid * CHUNK + k * 16 + lax.iota(jnp.int32, 16))
        else:
            slots_v[row, pl.ds(col, 16)] = jnp.full((16,), trash, jnp.int32)
            vals_v[row, pl.ds(col, 16)] = jnp.zeros((16,), jnp.int32)
    cps = [pltpu.async_copy(vals_v.at[c], compact_sp.at[slots_v.at[c]],
                            dsem) for c in range(7)]
    for cp in cps:
        cp.wait()


def _publish_count(wid, cnt, cnt_v, counts_sp):
    cnt_v[...] = jnp.full((16,), cnt, jnp.int32)
    pltpu.sync_copy(cnt_v, counts_sp.at[pl.ds(wid * 16, 16)])


def _read_counts(wid, counts_sp, counts_v):
    pltpu.sync_copy(counts_sp, counts_v)
    base = jnp.int32(0)
    tot = jnp.int32(0)
    for t in range(NT):
        c_t = counts_v[pl.ds(t * 16, 16)][0]
        base = base + jnp.where(t < wid, c_t, 0)
        tot = tot + c_t
    return base, tot


def _out_dma(wid, refs_v, refs_hbm, dsem, read=False):
    def _mk(v, h, dst, sl):
        vv = v.at[sl] if sl is not None else v
        if read:
            return pltpu.async_copy(h.at[dst], vv, dsem)
        return pltpu.async_copy(vv, h.at[dst], dsem)

    @pl.when(wid < NT - 1)
    def _full():
        dst = pl.ds(wid * OUT_PT, OUT_PT)
        cps = [_mk(v, h, dst, None) for v, h in zip(refs_v, refs_hbm)]
        for cp in cps:
            cp.wait()

    @pl.when(wid == NT - 1)
    def _tail():
        tl = OUT_PAD - (NT - 1) * OUT_PT
        dst = pl.ds((NT - 1) * OUT_PT, tl)
        sl = pl.ds(0, tl)
        cps = [_mk(v, h, dst, sl) for v, h in zip(refs_v, refs_hbm)]
        for cp in cps:
            cp.wait()


def _sc_body0(rm_hbm, j1_hbm, jl_hbm, bori_hbm, cm_hbm, borj_hbm, conf_hbm,
              b_hbm, i_hbm, j_hbm, m_hbm, v_hbm, tot_hbm,
              rm_c, j1_c, jl_c, bori_c, cm_v, borj_v,
              flags_v, ranks_v, fj_v, mrow_v, tie_v, row_v,
              slots_v, vals_v, cnt_v, counts_v, sel_v, fjt_v, mrt_v,
              outb_v, outi_v, outj_v, outm_v, outv_v,
              counts_sp, compact_sp, fj_sp, mrow_sp, dsem):
    wid = lax.axis_index("s")
    src = pl.ds(wid * CHUNK, CHUNK)

    @pl.when(wid < NTA)
    def _phase0():
        cnt = _sc_rows(0, wid, rm_hbm, j1_hbm, jl_hbm, bori_hbm, cm_hbm,
                       borj_hbm, conf_hbm, rm_c, j1_c, jl_c, bori_c, cm_v,
                       borj_v, flags_v, ranks_v, fj_v, mrow_v, tie_v, row_v,
                       dsem)
        pltpu.sync_copy(fj_v, fj_sp.at[src])
        pltpu.sync_copy(mrow_v, mrow_sp.at[src])
        _publish_count(wid, cnt, cnt_v, counts_sp)

    @pl.when(wid >= NTA)
    def _idle():
        _publish_count(wid, jnp.int32(0), cnt_v, counts_sp)

    plsc.subcore_barrier()
    base, tot = _read_counts(wid, counts_sp, counts_v)

    @pl.when(wid < NTA)
    def _sc():
        _sc_scatter(wid, base, flags_v, ranks_v, slots_v, vals_v,
                    compact_sp, dsem)
    cpt = [pltpu.async_copy(fj_sp, fjt_v, dsem),
           pltpu.async_copy(mrow_sp, mrt_v, dsem)]
    for cp in cpt:
        cp.wait()
    plsc.subcore_barrier()

    pltpu.sync_copy(compact_sp.at[pl.ds(wid * OUT_PT, OUT_PT)], sel_v)
    for k in range(OUT_PT // 16):
        sid = wid * OUT_PT + k * 16 + lax.iota(jnp.int32, 16)
        live = (sid < tot) & (sid < OUT_PAD)
        idx = jnp.where(live, sel_v[pl.ds(k * 16, 16)], 0)
        outb_v[pl.ds(k * 16, 16)] = jnp.zeros((16,), jnp.int32)
        outi_v[pl.ds(k * 16, 16)] = idx
        outj_v[pl.ds(k * 16, 16)] = plsc.load_gather(fjt_v, [idx])
        outm_v[pl.ds(k * 16, 16)] = plsc.load_gather(mrt_v, [idx])
        outv_v[pl.ds(k * 16, 16)] = live.astype(jnp.int32)
    _out_dma(wid, (outb_v, outi_v, outj_v, outm_v, outv_v),
             (b_hbm, i_hbm, j_hbm, m_hbm, v_hbm), dsem)

    @pl.when(wid == 0)
    def _tot():
        cnt_v[...] = jnp.full((16,), tot, jnp.int32)
        pltpu.sync_copy(cnt_v, tot_hbm)


def _sc_body1(rm_hbm, j1_hbm, jl_hbm, bori_hbm, cm_hbm, borj_hbm, conf_hbm,
              b0_hbm, i0_hbm, j0_hbm, m0_hbm, v0_hbm, tot0_hbm,
              b_hbm, i_hbm, j_hbm, m_hbm, v_hbm,
              rm_c, j1_c, jl_c, bori_c, cm_v, borj_v,
              flags_v, ranks_v, fj_v, mrow_v, tie_v, row_v,
              slots_v, vals_v, cnt_v, counts_v, sel_v, fjt_v, mrt_v,
              outb_v, outi_v, outj_v, outm_v, outv_v,
              p0b_v, p0i_v, p0j_v, p0m_v, p0v_v,
              counts_sp, compact_sp, fj_sp, mrow_sp, dsem):
    wid = lax.axis_index("s")
    src = pl.ds(wid * CHUNK, CHUNK)
    pltpu.sync_copy(tot0_hbm, cnt_v)
    tot0 = cnt_v[pl.ds(0, 16)][0]

    @pl.when(wid < NTA)
    def _phase0():
        cnt = _sc_rows(1, wid, rm_hbm, j1_hbm, jl_hbm, bori_hbm, cm_hbm,
                       borj_hbm, conf_hbm, rm_c, j1_c, jl_c, bori_c, cm_v,
                       borj_v, flags_v, ranks_v, fj_v, mrow_v, tie_v, row_v,
                       dsem)
        pltpu.sync_copy(fj_v, fj_sp.at[src])
        pltpu.sync_copy(mrow_v, mrow_sp.at[src])
        _publish_count(wid, cnt, cnt_v, counts_sp)

    @pl.when(wid >= NTA)
    def _idle():
        _publish_count(wid, jnp.int32(0), cnt_v, counts_sp)

    plsc.subcore_barrier()
    base1, tot1 = _read_counts(wid, counts_sp, counts_v)
    tot = tot0 + tot1

    @pl.when(wid < NTA)
    def _sc():
        _sc_scatter(wid, tot0 + base1, flags_v, ranks_v, slots_v, vals_v,
                    compact_sp, dsem)
    cpt = [pltpu.async_copy(fj_sp, fjt_v, dsem),
           pltpu.async_copy(mrow_sp, mrt_v, dsem)]
    for cp in cpt:
        cp.wait()
    _out_dma(wid, (p0b_v, p0i_v, p0j_v, p0m_v, p0v_v),
             (b0_hbm, i0_hbm, j0_hbm, m0_hbm, v0_hbm), dsem, read=True)
    plsc.subcore_barrier()

    pltpu.sync_copy(compact_sp.at[pl.ds(wid * OUT_PT, OUT_PT)], sel_v)
    for k in range(OUT_PT // 16):
        sl = pl.ds(k * 16, 16)
        sid = wid * OUT_PT + k * 16 + lax.iota(jnp.int32, 16)
        live0 = sid < tot0
        live1 = (sid >= tot0) & (sid < tot) & (sid < OUT_PAD)
        idx = jnp.where(live1, sel_v[sl], 0)
        j1g = plsc.load_gather(fjt_v, [idx])
        m1g = plsc.load_gather(mrt_v, [idx])
        outb_v[sl] = jnp.where(live0, p0b_v[sl], live1.astype(jnp.int32))
        outi_v[sl] = jnp.where(live0, p0i_v[sl], idx)
        outj_v[sl] = jnp.where(live0, p0j_v[sl],
                               jnp.where(live1, j1g, 0))
        outm_v[sl] = jnp.where(live0, p0m_v[sl],
                               jnp.where(live1, m1g, 0.0))
        outv_v[sl] = (live0 | live1).astype(jnp.int32)
    _out_dma(wid, (outb_v, outi_v, outj_v, outm_v, outv_v),
             (b_hbm, i_hbm, j_hbm, m_hbm, v_hbm), dsem)


def _border_vec():
    a = np.arange(L)
    h, w = a // W0C, a % W0C
    ok = (h >= BORDER_RM) & (h < H0C - BORDER_RM) & \
         (w >= BORDER_RM) & (w < W0C - BORDER_RM)
    return ok.astype(np.int32)


_BOR = _border_vec()
_JIO = np.arange(S, dtype=np.float32).reshape(1, 1, S)

_ROW_SCRATCH = [
    pltpu.VMEM((CHUNK,), jnp.float32),
    pltpu.VMEM((CHUNK,), jnp.int32),
    pltpu.VMEM((CHUNK,), jnp.int32),
    pltpu.VMEM((CHUNK,), jnp.int32),
    pltpu.VMEM((S,), jnp.float32),
    pltpu.VMEM((S,), jnp.int32),
    pltpu.VMEM((CHUNK,), jnp.int32),
    pltpu.VMEM((CHUNK,), jnp.int32),
    pltpu.VMEM((CHUNK,), jnp.int32),
    pltpu.VMEM((CHUNK,), jnp.float32),
    pltpu.VMEM((CHUNK,), jnp.int32),
    pltpu.VMEM((S,), jnp.float32),
    pltpu.VMEM((7, 128), jnp.int32),
    pltpu.VMEM((7, 128), jnp.int32),
    pltpu.VMEM((16,), jnp.int32),
    pltpu.VMEM((NT * 16,), jnp.int32),
    pltpu.VMEM((OUT_PT,), jnp.int32),
    pltpu.VMEM((L,), jnp.int32),
    pltpu.VMEM((L,), jnp.float32),
    pltpu.VMEM((OUT_PT,), jnp.int32),
    pltpu.VMEM((OUT_PT,), jnp.int32),
    pltpu.VMEM((OUT_PT,), jnp.int32),
    pltpu.VMEM((OUT_PT,), jnp.float32),
    pltpu.VMEM((OUT_PT,), jnp.int32),
]
_SHARED_SCRATCH = [
    pltpu.VMEM_SHARED((NT * 16,), jnp.int32),
    pltpu.VMEM_SHARED((CSIZE,), jnp.int32),
    pltpu.VMEM_SHARED((L,), jnp.int32),
    pltpu.VMEM_SHARED((L,), jnp.float32),
]
_OUT5 = [
    jax.ShapeDtypeStruct((OUT_PAD,), jnp.int32),
    jax.ShapeDtypeStruct((OUT_PAD,), jnp.int32),
    jax.ShapeDtypeStruct((OUT_PAD,), jnp.int32),
    jax.ShapeDtypeStruct((OUT_PAD,), jnp.float32),
    jax.ShapeDtypeStruct((OUT_PAD,), jnp.int32),
]


@functools.lru_cache(maxsize=1)
def _make_sc_calls():
  mesh = plsc.VectorSubcoreMesh(
      core_axis_name="c", subcore_axis_name="s", num_cores=1, num_subcores=NT)
  sc0 = pl.kernel(
      _sc_body0,
      out_type=list(_OUT5) + [jax.ShapeDtypeStruct((16,), jnp.int32)],
      mesh=mesh,
      scratch_types=list(_ROW_SCRATCH) + list(_SHARED_SCRATCH)
      + [pltpu.SemaphoreType.DMA],
      compiler_params=pltpu.CompilerParams(needs_layout_passes=False),
  )
  sc1 = pl.kernel(
      _sc_body1,
      out_type=list(_OUT5),
      mesh=mesh,
      scratch_types=list(_ROW_SCRATCH)
      + [pltpu.VMEM((OUT_PT,), jnp.int32),
         pltpu.VMEM((OUT_PT,), jnp.int32),
         pltpu.VMEM((OUT_PT,), jnp.int32),
         pltpu.VMEM((OUT_PT,), jnp.float32),
         pltpu.VMEM((OUT_PT,), jnp.int32)]
      + list(_SHARED_SCRATCH) + [pltpu.SemaphoreType.DMA],
      compiler_params=pltpu.CompilerParams(needs_layout_passes=False),
  )
  return sc0, sc1


def _tc_call(conf, jio, bat):
    prs = pl.BlockSpec((1, 1, RB), lambda r: (r, 0, 0))
    return pl.pallas_call(
        _merged_body,
        grid=(NR,),
        in_specs=[pl.BlockSpec((1, RB, S), lambda r, b=bat: (b, r, 0)),
                  pl.BlockSpec((1, 1, S), lambda r: (0, 0, 0))],
        out_specs=[pl.BlockSpec((1, 1, S), lambda r: (0, 0, 0)),
                   prs, prs, prs],
        out_shape=[jax.ShapeDtypeStruct((1, 1, S), jnp.float32),
                   jax.ShapeDtypeStruct((NR, 1, RB), jnp.float32),
                   jax.ShapeDtypeStruct((NR, 1, RB), jnp.int32),
                   jax.ShapeDtypeStruct((NR, 1, RB), jnp.int32)],
    )(conf, jio)


def kernel(conf_matrix, h0c, w0c, h1c, w1c):
    conf = conf_matrix
    jio = jnp.asarray(_JIO)
    bor = jnp.asarray(_BOR)
    conf2 = conf.reshape(NROWS, S)
    sc0, sc1 = _make_sc_calls()

    cm0, rm0, j10, jl0 = _tc_call(conf, jio, 0)
    cm1, rm1, j11, jl1 = _tc_call(conf, jio, 1)
    b0, i0, j0, m0, v0, tot0 = sc0(rm0, j10, jl0, bor, cm0, bor, conf2)
    bi, ii, jj, mm, vv = sc1(rm1, j11, jl1, bor, cm1, bor, conf2,
                             b0, i0, j0, m0, v0, tot0)

    resid = ((jnp.asarray(h0c) - H0C) + (jnp.asarray(w0c) - W0C)
             + (jnp.asarray(h1c) - H1C)
             + (jnp.asarray(w1c) - W1C)).astype(jnp.float32)
    return (bi, ii, jj, mm + resid, vv.astype(bool))


# speedup vs baseline: 1.0598x; 1.0598x over previous
import functools

import jax
import jax.numpy as jnp
import numpy as np
from jax import lax
from jax.experimental import pallas as pl
from jax.experimental.pallas import tpu as pltpu
from jax.experimental.pallas import tpu_sc as plsc

THR = 0.2
BORDER_RM = 2
NUM_MATCHES = 5000
B, H0C, W0C, H1C, W1C = 2, 60, 80, 60, 80
L = H0C * W0C
S = H1C * W1C
RB = 800
NR = L // RB
NROWS = B * L

NT = 16
NTA = 12
CHUNK = 800
NV = CHUNK // 16
OUT_PAD = 5000
OUT_PT = 320
CSIZE = 5248
TRASH = OUT_PAD + 16

_THR_GE = float(np.nextafter(np.float32(THR), np.float32(1.0)))


def _merged_body(conf_ref, jio_ref, cmax_ref, rm_ref, j1_ref, jl_ref):
    r = pl.program_id(1)
    x = conf_ref[0]
    rm = jnp.max(x, axis=1, keepdims=True)
    pmax = jnp.max(conf_ref[...], axis=1, keepdims=True)

    @pl.when(r == 0)
    def _():
        cmax_ref[...] = pmax

    @pl.when(r != 0)
    def _():
        cmax_ref[...] = jnp.maximum(cmax_ref[...], pmax)

    ge = x >= rm
    jio = jio_ref[0]
    j1 = jnp.min(jnp.where(ge, jio, jnp.float32(S)), axis=1)
    jl = jnp.max(jnp.where(ge, jio, jnp.float32(-1)), axis=1)
    rm_ref[...] = rm.reshape(1, 1, RB)
    j1_ref[...] = j1.astype(jnp.int32).reshape(1, 1, RB)
    jl_ref[...] = jl.astype(jnp.int32).reshape(1, 1, RB)


def _sc_body(rm_hbm, j1_hbm, jl_hbm, bori_hbm, cm_hbm, borj_hbm, conf_hbm,
             b_hbm, i_hbm, j_hbm, m_hbm, v_hbm,
             rm_c, j1_c, jl_c, bori_c, cm_v, borj_v,
             flags_v, ranks_v, fj_v, mrow_v, tie_v, row_v,
             slots_v, vals_v, cnt_v, counts_v, sel_v, fjt_v, mrt_v,
             outb_v, outi_v, outj_v, outm_v, outv_v,
             counts_sp, compact_sp, fj_sp, mrow_sp, dsem):
    wid = lax.axis_index("s")
    src = pl.ds(wid * CHUNK, CHUNK)

    @pl.when(wid < NTA)
    def _phase0():
        cps = [pltpu.async_copy(rm_hbm.at[wid, 0], rm_c, dsem),
               pltpu.async_copy(j1_hbm.at[wid, 0], j1_c, dsem),
               pltpu.async_copy(jl_hbm.at[wid, 0], jl_c, dsem),
               pltpu.async_copy(bori_hbm.at[src], bori_c, dsem),
               pltpu.async_copy(cm_hbm.at[0, 0], cm_v.at[pl.ds(0, S)], dsem),
               pltpu.async_copy(cm_hbm.at[1, 0], cm_v.at[pl.ds(S, S)], dsem),
               pltpu.async_copy(borj_hbm, borj_v, dsem)]
        for cp in cps:
            cp.wait()

        ntie = jnp.int32(0)
        for k in range(NV):
            sl = pl.ds(k * 16, 16)
            rmv = rm_c[sl]
            j1v = j1_c[sl]
            jlv = jl_c[sl]
            grow = wid * CHUNK + k * 16 + lax.iota(jnp.int32, 16)
            bb = (grow >= L).astype(jnp.int32)
            c1 = plsc.load_gather(cm_v, [bb * S + j1v])
            bj1 = plsc.load_gather(borj_v, [j1v])
            rowok = (bori_c[sl] > 0) & (rmv >= _THR_GE)
            tie = rowok & (jlv > j1v)
            okrow = rowok & (jlv == j1v) & (c1 == rmv) & (bj1 > 0)
            flags_v[sl] = okrow.astype(jnp.int32)
            fj_v[sl] = jnp.where(okrow, j1v, 0)
            mrow_v[sl] = jnp.where(okrow, rmv, 0.0)
            ti = tie.astype(jnp.int32)
            plsc.store_scatter(tie_v, [ntie + plsc.cumsum(ti) - ti],
                               k * 16 + lax.iota(jnp.int32, 16), mask=tie)
            ntie = ntie + jnp.sum(ti)

        def _refine(t, carry):
            r = plsc.load_gather(tie_v, [jnp.full((16,), t, jnp.int32)])[0]
            grow = wid * CHUNK + r
            pltpu.sync_copy(conf_hbm.at[grow], row_v)
            rms = plsc.load_gather(rm_c, [jnp.full((16,), r, jnp.int32)])[0]
            rmf = jnp.full((16,), rms, jnp.float32)
            cmoff = jnp.where(grow >= L, S, 0)

            def _scan(k, vmin):
                cv = row_v[pl.ds(k * 16, 16)]
                cmv = cm_v[pl.ds(cmoff + k * 16, 16)]
                bjv = borj_v[pl.ds(k * 16, 16)]
                jv = k * 16 + lax.iota(jnp.int32, 16)
                hit = (cv == rmf) & (cmv == rmf) & (bjv > 0)
                return jnp.minimum(vmin, jnp.where(hit, jv, S))

            vmin = lax.fori_loop(0, S // 16, _scan,
                                 jnp.full((16,), S, jnp.int32))
            fjs = jnp.min(vmin)
            found = fjs < S
            base = (r // 16) * 16
            eq = lax.iota(jnp.int32, 16) == (r - base)
            bsl = pl.ds(base, 16)
            flags_v[bsl] = jnp.where(eq, found.astype(jnp.int32),
                                     flags_v[bsl])
            fnd = eq & found
            fj_v[bsl] = jnp.where(fnd, fjs, fj_v[bsl])
            mrow_v[bsl] = jnp.where(fnd, rms, mrow_v[bsl])
            return carry

        lax.fori_loop(0, ntie, _refine, jnp.int32(0))

        cnt = jnp.int32(0)
        for k in range(NV):
            sl = pl.ds(k * 16, 16)
            f = flags_v[sl]
            ranks_v[sl] = cnt + (plsc.cumsum(f) - f)
            cnt = cnt + jnp.sum(f)
        pltpu.sync_copy(fj_v, fj_sp.at[src])
        pltpu.sync_copy(mrow_v, mrow_sp.at[src])
        cnt_v[...] = jnp.full((16,), cnt, jnp.int32)
        pltpu.sync_copy(cnt_v, counts_sp.at[pl.ds(wid * 16, 16)])

    @pl.when(wid >= NTA)
    def _idle():
        cnt_v[...] = jnp.zeros((16,), jnp.int32)
        pltpu.sync_copy(cnt_v, counts_sp.at[pl.ds(wid * 16, 16)])

    plsc.subcore_barrier()

    pltpu.sync_copy(counts_sp, counts_v)
    base = jnp.int32(0)
    tot = jnp.int32(0)
    for t in range(NT):
        c_t = counts_v[pl.ds(t * 16, 16)][0]
        base = base + jnp.where(t < wid, c_t, 0)
        tot = tot + c_t

    @pl.when(wid < NTA)
    def _scatter():
        trash = TRASH + wid
        for k in range(56):
            row, col = k // 8, (k % 8) * 16
            if k < NV:
                f = flags_v[pl.ds(k * 16, 16)]
                slot = base + ranks_v[pl.ds(k * 16, 16)]
                ok = (f > 0) & (slot < OUT_PAD)
                slots_v[row, pl.ds(col, 16)] = jnp.where(ok, slot, trash)
                vals_v[row, pl.ds(col, 16)] = (
                    wid * CHUNK + k * 16 + lax.iota(jnp.int32, 16))
            else:
                slots_v[row, pl.ds(col, 16)] = jnp.full((16,), trash,
                                                        jnp.int32)
                vals_v[row, pl.ds(col, 16)] = jnp.zeros((16,), jnp.int32)
        cps = [pltpu.async_copy(vals_v.at[c], compact_sp.at[slots_v.at[c]],
                                dsem) for c in range(7)]
        for cp in cps:
            cp.wait()

    cpt = [pltpu.async_copy(fj_sp, fjt_v, dsem),
           pltpu.async_copy(mrow_sp, mrt_v, dsem)]
    for cp in cpt:
        cp.wait()
    plsc.subcore_barrier()

    pltpu.sync_copy(compact_sp.at[pl.ds(wid * OUT_PT, OUT_PT)], sel_v)
    for k in range(OUT_PT // 16):
        sid = wid * OUT_PT + k * 16 + lax.iota(jnp.int32, 16)
        live = (sid < tot) & (sid < OUT_PAD)
        idx = jnp.where(live, sel_v[pl.ds(k * 16, 16)], 0)
        jv = plsc.load_gather(fjt_v, [idx])
        mv = plsc.load_gather(mrt_v, [idx])
        bv = (idx >= L).astype(jnp.int32)
        outb_v[pl.ds(k * 16, 16)] = bv
        outi_v[pl.ds(k * 16, 16)] = idx - bv * L
        outj_v[pl.ds(k * 16, 16)] = jv
        outm_v[pl.ds(k * 16, 16)] = mv
        outv_v[pl.ds(k * 16, 16)] = live.astype(jnp.int32)
    @pl.when(wid < NT - 1)
    def _out_full():
        dst = pl.ds(wid * OUT_PT, OUT_PT)
        cps = [pltpu.async_copy(outb_v, b_hbm.at[dst], dsem),
               pltpu.async_copy(outi_v, i_hbm.at[dst], dsem),
               pltpu.async_copy(outj_v, j_hbm.at[dst], dsem),
               pltpu.async_copy(outm_v, m_hbm.at[dst], dsem),
               pltpu.async_copy(outv_v, v_hbm.at[dst], dsem)]
        for cp in cps:
            cp.wait()

    @pl.when(wid == NT - 1)
    def _out_tail():
        tl = OUT_PAD - (NT - 1) * OUT_PT
        dst = pl.ds((NT - 1) * OUT_PT, tl)
        sl = pl.ds(0, tl)
        cps = [pltpu.async_copy(outb_v.at[sl], b_hbm.at[dst], dsem),
               pltpu.async_copy(outi_v.at[sl], i_hbm.at[dst], dsem),
               pltpu.async_copy(outj_v.at[sl], j_hbm.at[dst], dsem),
               pltpu.async_copy(outm_v.at[sl], m_hbm.at[dst], dsem),
               pltpu.async_copy(outv_v.at[sl], v_hbm.at[dst], dsem)]
        for cp in cps:
            cp.wait()


def _border_vec():
    a = np.arange(L)
    h, w = a // W0C, a % W0C
    ok = (h >= BORDER_RM) & (h < H0C - BORDER_RM) & \
         (w >= BORDER_RM) & (w < W0C - BORDER_RM)
    return ok.astype(np.int32)


_BOR = _border_vec()
_BORI_FLAT = np.tile(_BOR, B)
_JIO = np.arange(S, dtype=np.float32).reshape(1, 1, S)


@functools.lru_cache(maxsize=1)
def _make_sc_call():
  mesh = plsc.VectorSubcoreMesh(
      core_axis_name="c", subcore_axis_name="s", num_cores=1, num_subcores=NT)
  return pl.kernel(
    _sc_body,
    out_type=[
        jax.ShapeDtypeStruct((OUT_PAD,), jnp.int32),
        jax.ShapeDtypeStruct((OUT_PAD,), jnp.int32),
        jax.ShapeDtypeStruct((OUT_PAD,), jnp.int32),
        jax.ShapeDtypeStruct((OUT_PAD,), jnp.float32),
        jax.ShapeDtypeStruct((OUT_PAD,), jnp.int32),
    ],
    mesh=mesh,
    scratch_types=[
        pltpu.VMEM((CHUNK,), jnp.float32),
        pltpu.VMEM((CHUNK,), jnp.int32),
        pltpu.VMEM((CHUNK,), jnp.int32),
        pltpu.VMEM((CHUNK,), jnp.int32),
        pltpu.VMEM((NROWS,), jnp.float32),
        pltpu.VMEM((S,), jnp.int32),
        pltpu.VMEM((CHUNK,), jnp.int32),
        pltpu.VMEM((CHUNK,), jnp.int32),
        pltpu.VMEM((CHUNK,), jnp.int32),
        pltpu.VMEM((CHUNK,), jnp.float32),
        pltpu.VMEM((CHUNK,), jnp.int32),
        pltpu.VMEM((S,), jnp.float32),
        pltpu.VMEM((7, 128), jnp.int32),
        pltpu.VMEM((7, 128), jnp.int32),
        pltpu.VMEM((16,), jnp.int32),
        pltpu.VMEM((NT * 16,), jnp.int32),
        pltpu.VMEM((OUT_PT,), jnp.int32),
        pltpu.VMEM((NROWS,), jnp.int32),
        pltpu.VMEM((NROWS,), jnp.float32),
        pltpu.VMEM((OUT_PT,), jnp.int32),
        pltpu.VMEM((OUT_PT,), jnp.int32),
        pltpu.VMEM((OUT_PT,), jnp.int32),
        pltpu.VMEM((OUT_PT,), jnp.float32),
        pltpu.VMEM((OUT_PT,), jnp.int32),
        pltpu.VMEM_SHARED((NT * 16,), jnp.int32),
        pltpu.VMEM_SHARED((CSIZE,), jnp.int32),
        pltpu.VMEM_SHARED((NROWS,), jnp.int32),
        pltpu.VMEM_SHARED((NROWS,), jnp.float32),
        pltpu.SemaphoreType.DMA,
    ],
    compiler_params=pltpu.CompilerParams(needs_layout_passes=False),
  )


def kernel(conf_matrix, h0c, w0c, h1c, w1c):
    conf = conf_matrix
    prs = pl.BlockSpec((1, 1, RB), lambda b, r: (b * NR + r, 0, 0))
    prt = jax.ShapeDtypeStruct((B * NR, 1, RB), jnp.int32)
    prtf = jax.ShapeDtypeStruct((B * NR, 1, RB), jnp.float32)

    colmax, rm3, j13, jl3 = pl.pallas_call(
        _merged_body,
        grid=(B, NR),
        in_specs=[pl.BlockSpec((1, RB, S), lambda b, r: (b, r, 0)),
                  pl.BlockSpec((1, 1, S), lambda b, r: (0, 0, 0))],
        out_specs=[pl.BlockSpec((1, 1, S), lambda b, r: (b, 0, 0)),
                   prs, prs, prs],
        out_shape=[jax.ShapeDtypeStruct((B, 1, S), jnp.float32),
                   prtf, prt, prt],
    )(conf, jnp.asarray(_JIO))

    bi, ii, jj, mm, vv = _make_sc_call()(
        rm3, j13, jl3,
        jnp.asarray(_BORI_FLAT), colmax,
        jnp.asarray(_BOR), conf.reshape(NROWS, S))

    resid = ((jnp.asarray(h0c) - H0C) + (jnp.asarray(w0c) - W0C)
             + (jnp.asarray(h1c) - H1C)
             + (jnp.asarray(w1c) - W1C)).astype(jnp.float32)
    return (bi, ii, jj, mm + resid, vv.astype(bool))
